# baseline (speedup 1.0000x reference)
import jax
import jax.numpy as jnp
import numpy as np
from jax import lax
from jax.experimental import pallas as pl
from jax.experimental.pallas import tpu as pltpu
from jax.experimental.pallas import tpu_sc as plsc


_B = 524288
_D = 3
_C = 2
_L = 16
_OUT_D = _L * _C
_NW = 32
_W = _B // _NW
_P = 1024
_NCH = _W // _P
_NSUB = _P // 16

_N_ROWS = 7131240
_MASK = (1 << 19) - 1
_PRIME1 = np.int32(np.uint32(2654435761).view(np.int32))
_PRIME2 = np.int32(805459861)

_DENSE = [
    (15.0, 17, 17 * 17, 0),
    (31.0, 33, 33 * 33, 4920),
    (63.0, 65, 65 * 65, 40864),
]
_HASH_BASE0 = 315496
_HASH_SCALE0 = 127.0
_HASH_STRIDE = 1 << 19

_CCH = 2048
_NFULL = _N_ROWS // _CCH
_CTAIL = _N_ROWS - _NFULL * _CCH
_CREM = _NFULL % _NW


def _conv_body(ch0_hbm, ch1_hbm, flat_hbm, c0, c1, ibuf, tbuf):
    wid = lax.axis_index("s") * 2 + lax.axis_index("c")
    iota = lax.iota(jnp.int32, 16)
    iota2 = iota * 2

    def interleave(n16, src0, src1, dst):
        def body(i, _):
            o16 = i * 16
            v0 = src0[pl.ds(o16, 16)]
            v1 = src1[pl.ds(o16, 16)]
            pos = iota2 + o16 * 2
            plsc.store_scatter(dst, [pos], v0)
            plsc.store_scatter(dst, [pos + 1], v1)
            return 0
        lax.fori_loop(0, n16, body, 0, unroll=False)

    ntrips = 108 + (wid < _CREM).astype(jnp.int32)

    def chunk(k, _):
        b = (k * _NW + wid) * _CCH
        pltpu.sync_copy(ch0_hbm.at[pl.ds(b, _CCH)], c0)
        pltpu.sync_copy(ch1_hbm.at[pl.ds(b, _CCH)], c1)
        interleave(_CCH // 16, c0, c1, ibuf)
        pltpu.sync_copy(ibuf, flat_hbm.at[pl.ds(b * 2, _CCH * 2)])
        return 0

    lax.fori_loop(0, ntrips, chunk, 0, unroll=False)

    @pl.when(wid == _NW - 1)
    def _():
        b = _NFULL * _CCH
        pltpu.sync_copy(ch0_hbm.at[pl.ds(b, _CTAIL)], c0.at[pl.ds(0, _CTAIL)])
        pltpu.sync_copy(ch1_hbm.at[pl.ds(b, _CTAIL)], c1.at[pl.ds(0, _CTAIL)])
        interleave(_CTAIL // 16, c0, c1, tbuf)
        o16 = (_CTAIL // 16) * 16
        v0 = c0[pl.ds(o16, 16)]
        v1 = c1[pl.ds(o16, 16)]
        pos = iota2 + o16 * 2
        msk = iota < (_CTAIL - o16)
        plsc.store_scatter(tbuf, [pos], v0, mask=msk)
        plsc.store_scatter(tbuf, [pos + 1], v1, mask=msk)
        pltpu.sync_copy(tbuf.at[pl.ds(0, _CTAIL * 2)],
                        flat_hbm.at[pl.ds(b * 2, _CTAIL * 2)])


def _main_body(x_hbm, y_hbm, z_hbm, emb_hbm, out_hbm, *scratch):
    xb, yb, zb = scratch[0:3]
    frac_bufs = (scratch[3:6], scratch[6:9])
    idx_bufs = (scratch[9:17], scratch[17:25])
    row_bufs = (scratch[25:33], scratch[33:41])
    outb = scratch[41]
    sems = (scratch[42], scratch[43])

    wid = lax.axis_index("s") * 2 + lax.axis_index("c")

    iota = lax.iota(jnp.int32, 16)
    iota2 = iota * 2
    iota32 = iota * 32

    levels = []
    for l, (scale, r, r2, base) in enumerate(_DENSE):
        levels.append((scale, base, (r, r2, False)))
    for l in range(3, 12):
        levels.append((2.0 ** l * 16 - 1, _HASH_BASE0 + (l - 3) * _HASH_STRIDE,
                       None))
    levels.append((65535.0, 5034088, (65537, 131073, True)))
    levels.append((131071.0, 5558376, (131073, 262145, True)))
    levels.append((262143.0, 6082664, None))
    levels.append((524287.0, 6606952, None))

    def compute_idx(lv, bs):
        scale, base, lin = levels[lv]
        s_f = jnp.float32(scale)
        base = jnp.int32(base)
        fxb, fyb, fzb = frac_bufs[bs]
        idxs = idx_bufs[bs]

        def idx_body(i, _):
            o16 = i * 16
            x = xb[pl.ds(o16, 16)]
            y = yb[pl.ds(o16, 16)]
            z = zb[pl.ds(o16, 16)]

            px = x * s_f + 0.5
            py = y * s_f + 0.5
            pz = z * s_f + 0.5
            ix0 = px.astype(jnp.int32)
            iy0 = py.astype(jnp.int32)
            iz0 = pz.astype(jnp.int32)
            fxb[pl.ds(o16, 16)] = px - ix0.astype(jnp.float32)
            fyb[pl.ds(o16, 16)] = py - iy0.astype(jnp.float32)
            fzb[pl.ds(o16, 16)] = pz - iz0.astype(jnp.float32)
            ix1 = ix0 + 1
            if lin is None:
                hy0 = iy0 * _PRIME1
                hy1 = hy0 + _PRIME1
                hz0 = iz0 * _PRIME2
                hz1 = hz0 + _PRIME2
                terms = []
                for c in range(8):
                    xx = ix1 if (c & 1) else ix0
                    hy = hy1 if (c & 2) else hy0
                    hz = hz1 if (c & 4) else hz0
                    terms.append(((xx ^ hy ^ hz) & _MASK) + base)
            else:
                m1, m2, masked = lin
                my0 = iy0 * m1
                my1 = my0 + m1
                mz0 = iz0 * m2
                mz1 = mz0 + m2
                terms = []
                for c in range(8):
                    xx = ix1 if (c & 1) else ix0
                    my = my1 if (c & 2) else my0
                    mz = mz1 if (c & 4) else mz0
                    t = xx + my + mz
                    if masked:
                        t = t & _MASK
                    terms.append(t + base)
            p0 = iota2 + i * 32
            for c in range(8):
                t2 = terms[c] + terms[c]
                plsc.store_scatter(idxs[c], [p0], t2)
                plsc.store_scatter(idxs[c], [p0 + 1], t2 + 1)
            return 0

        lax.fori_loop(0, _NSUB, idx_body, 0, unroll=False)

    def fire(bs):
        return [
            pltpu.async_copy(emb_hbm.at[idx_bufs[bs][c]], row_bufs[bs][c],
                             sems[bs])
            for c in range(8)
        ]

    def accumulate(lv, bs):
        col2 = lv * 2
        fxb, fyb, fzb = frac_bufs[bs]
        rows = row_bufs[bs]

        def acc_body(i, _):
            o16 = i * 16
            fx = fxb[pl.ds(o16, 16)]
            fy = fyb[pl.ds(o16, 16)]
            fz = fzb[pl.ds(o16, 16)]
            gx = 1.0 - fx
            gy = 1.0 - fy
            gz = 1.0 - fz
            wxy = [gx * gy, fx * gy, gx * fy, fx * fy]
            g0 = iota2 + i * 32
            acc0 = jnp.zeros((16,), jnp.float32)
            acc1 = jnp.zeros((16,), jnp.float32)
            for c in range(8):
                w = wxy[c & 3] * (fz if (c & 4) else gz)
                r0 = plsc.load_gather(rows[c], [g0])
                r1 = plsc.load_gather(rows[c], [g0 + 1])
                acc0 = acc0 + w * r0
                acc1 = acc1 + w * r1
            o0 = iota32 + (i * 512 + col2)
            plsc.store_scatter(outb, [o0], acc0)
            plsc.store_scatter(outb, [o0 + 1], acc1)
            return 0

        lax.fori_loop(0, _NSUB, acc_body, 0, unroll=False)

    def chunk_body(t, _):
        pt_base = wid * _W + t * _P
        pltpu.sync_copy(x_hbm.at[pl.ds(pt_base, _P)], xb)
        pltpu.sync_copy(y_hbm.at[pl.ds(pt_base, _P)], yb)
        pltpu.sync_copy(z_hbm.at[pl.ds(pt_base, _P)], zb)

        compute_idx(0, 0)
        pending = fire(0)
        for lv in range(_L):
            bs = lv % 2
            if lv + 1 < _L:
                compute_idx(lv + 1, 1 - bs)
                nxt = fire(1 - bs)
            for cp in pending:
                cp.wait()
            accumulate(lv, bs)
            if lv + 1 < _L:
                pending = nxt

        pltpu.sync_copy(outb, out_hbm.at[pl.ds(pt_base * _OUT_D, _P * _OUT_D)])
        return 0

    lax.fori_loop(0, _NCH, chunk_body, 0, unroll=False)


@jax.jit
def _grid_encode_sc(inputs, embeddings):
    conv = pl.kernel(
        _conv_body,
        out_type=jax.ShapeDtypeStruct((_N_ROWS * _C,), jnp.float32),
        mesh=plsc.VectorSubcoreMesh(core_axis_name="c", subcore_axis_name="s"),
        scratch_types=[
            pltpu.VMEM((_CCH,), jnp.float32),
            pltpu.VMEM((_CCH,), jnp.float32),
            pltpu.VMEM((_CCH * 2,), jnp.float32),
            pltpu.VMEM(((_CTAIL // 16 + 1) * 32,), jnp.float32),
        ],
        compiler_params=pltpu.CompilerParams(needs_layout_passes=False),
    )
    emb_flat = conv(embeddings[:, 0], embeddings[:, 1])

    scratch = [pltpu.VMEM((_P,), jnp.float32) for _ in range(3)]
    scratch += [pltpu.VMEM((_P,), jnp.float32) for _ in range(6)]
    scratch += [pltpu.VMEM((_P * _C,), jnp.int32) for _ in range(16)]
    scratch += [pltpu.VMEM((_P * _C,), jnp.float32) for _ in range(16)]
    scratch += [
        pltpu.VMEM((_P * _OUT_D,), jnp.float32),
        pltpu.SemaphoreType.DMA,
        pltpu.SemaphoreType.DMA,
    ]
    kern = pl.kernel(
        _main_body,
        out_type=jax.ShapeDtypeStruct((_B * _OUT_D,), jnp.float32),
        mesh=plsc.VectorSubcoreMesh(core_axis_name="c", subcore_axis_name="s"),
        scratch_types=scratch,
        compiler_params=pltpu.CompilerParams(needs_layout_passes=False),
    )
    flat = kern(inputs[:, 0], inputs[:, 1], inputs[:, 2], emb_flat)
    return flat.reshape(_B, _OUT_D)


def kernel(inputs, embeddings, offsets):
    del offsets
    return _grid_encode_sc(inputs, embeddings)


# speedup vs baseline: 1.0025x; 1.0025x over previous
import jax
import jax.numpy as jnp
import numpy as np
from jax import lax
from jax.experimental import pallas as pl
from jax.experimental.pallas import tpu as pltpu
from jax.experimental.pallas import tpu_sc as plsc


_B = 524288
_D = 3
_C = 2
_L = 16
_OUT_D = _L * _C
_NW = 32
_W = _B // _NW
_P = 512
_NCH = _W // _P
_NSUB = _P // 16

_N_ROWS = 7131240
_MASK = (1 << 19) - 1
_PRIME1 = np.int32(np.uint32(2654435761).view(np.int32))
_PRIME2 = np.int32(805459861)

_DENSE = [
    (15.0, 17, 17 * 17, 0),
    (31.0, 33, 33 * 33, 4920),
    (63.0, 65, 65 * 65, 40864),
]
_HASH_BASE0 = 315496
_HASH_SCALE0 = 127.0
_HASH_STRIDE = 1 << 19

_CCH = 2048
_NFULL = _N_ROWS // _CCH
_CTAIL = _N_ROWS - _NFULL * _CCH
_CREM = _NFULL % _NW


def _conv_body(ch0_hbm, ch1_hbm, flat_hbm, c0, c1, ibuf, tbuf):
    wid = lax.axis_index("s") * 2 + lax.axis_index("c")
    iota = lax.iota(jnp.int32, 16)
    iota2 = iota * 2

    def interleave(n16, src0, src1, dst):
        def body(i, _):
            o16 = i * 16
            v0 = src0[pl.ds(o16, 16)]
            v1 = src1[pl.ds(o16, 16)]
            pos = iota2 + o16 * 2
            plsc.store_scatter(dst, [pos], v0)
            plsc.store_scatter(dst, [pos + 1], v1)
            return 0
        lax.fori_loop(0, n16, body, 0, unroll=False)

    ntrips = 108 + (wid < _CREM).astype(jnp.int32)

    def chunk(k, _):
        b = (k * _NW + wid) * _CCH
        pltpu.sync_copy(ch0_hbm.at[pl.ds(b, _CCH)], c0)
        pltpu.sync_copy(ch1_hbm.at[pl.ds(b, _CCH)], c1)
        interleave(_CCH // 16, c0, c1, ibuf)
        pltpu.sync_copy(ibuf, flat_hbm.at[pl.ds(b * 2, _CCH * 2)])
        return 0

    lax.fori_loop(0, ntrips, chunk, 0, unroll=False)

    @pl.when(wid == _NW - 1)
    def _():
        b = _NFULL * _CCH
        pltpu.sync_copy(ch0_hbm.at[pl.ds(b, _CTAIL)], c0.at[pl.ds(0, _CTAIL)])
        pltpu.sync_copy(ch1_hbm.at[pl.ds(b, _CTAIL)], c1.at[pl.ds(0, _CTAIL)])
        interleave(_CTAIL // 16, c0, c1, tbuf)
        o16 = (_CTAIL // 16) * 16
        v0 = c0[pl.ds(o16, 16)]
        v1 = c1[pl.ds(o16, 16)]
        pos = iota2 + o16 * 2
        msk = iota < (_CTAIL - o16)
        plsc.store_scatter(tbuf, [pos], v0, mask=msk)
        plsc.store_scatter(tbuf, [pos + 1], v1, mask=msk)
        pltpu.sync_copy(tbuf.at[pl.ds(0, _CTAIL * 2)],
                        flat_hbm.at[pl.ds(b * 2, _CTAIL * 2)])


def _main_body(x_hbm, y_hbm, z_hbm, emb_hbm, out_hbm, *scratch):
    xb, yb, zb = scratch[0:3]
    frac_bufs = (scratch[3:6], scratch[6:9])
    idx_bufs = (scratch[9:17], scratch[17:25])
    row_bufs = (scratch[25:33], scratch[33:41])
    outb = scratch[41]
    sems = (scratch[42], scratch[43])

    wid = lax.axis_index("s") * 2 + lax.axis_index("c")

    iota = lax.iota(jnp.int32, 16)
    iota2 = iota * 2
    iota32 = iota * 32

    levels = []
    for l, (scale, r, r2, base) in enumerate(_DENSE):
        levels.append((scale, base, (r, r2, False)))
    for l in range(3, 12):
        levels.append((2.0 ** l * 16 - 1, _HASH_BASE0 + (l - 3) * _HASH_STRIDE,
                       None))
    levels.append((65535.0, 5034088, (65537, 131073, True)))
    levels.append((131071.0, 5558376, (131073, 262145, True)))
    levels.append((262143.0, 6082664, None))
    levels.append((524287.0, 6606952, None))

    def compute_idx(lv, bs):
        scale, base, lin = levels[lv]
        s_f = jnp.float32(scale)
        base = jnp.int32(base)
        fxb, fyb, fzb = frac_bufs[bs]
        idxs = idx_bufs[bs]

        def idx_body(i, _):
            o16 = i * 16
            x = xb[pl.ds(o16, 16)]
            y = yb[pl.ds(o16, 16)]
            z = zb[pl.ds(o16, 16)]

            px = x * s_f + 0.5
            py = y * s_f + 0.5
            pz = z * s_f + 0.5
            ix0 = px.astype(jnp.int32)
            iy0 = py.astype(jnp.int32)
            iz0 = pz.astype(jnp.int32)
            fxb[pl.ds(o16, 16)] = px - ix0.astype(jnp.float32)
            fyb[pl.ds(o16, 16)] = py - iy0.astype(jnp.float32)
            fzb[pl.ds(o16, 16)] = pz - iz0.astype(jnp.float32)
            ix1 = ix0 + 1
            if lin is None:
                hy0 = iy0 * _PRIME1
                hy1 = hy0 + _PRIME1
                hz0 = iz0 * _PRIME2
                hz1 = hz0 + _PRIME2
                terms = []
                for c in range(8):
                    xx = ix1 if (c & 1) else ix0
                    hy = hy1 if (c & 2) else hy0
                    hz = hz1 if (c & 4) else hz0
                    terms.append(((xx ^ hy ^ hz) & _MASK) + base)
            else:
                m1, m2, masked = lin
                my0 = iy0 * m1
                my1 = my0 + m1
                mz0 = iz0 * m2
                mz1 = mz0 + m2
                terms = []
                for c in range(8):
                    xx = ix1 if (c & 1) else ix0
                    my = my1 if (c & 2) else my0
                    mz = mz1 if (c & 4) else mz0
                    t = xx + my + mz
                    if masked:
                        t = t & _MASK
                    terms.append(t + base)
            p0 = iota2 + i * 32
            for c in range(8):
                t2 = terms[c] + terms[c]
                plsc.store_scatter(idxs[c], [p0], t2)
                plsc.store_scatter(idxs[c], [p0 + 1], t2 + 1)
            return 0

        lax.fori_loop(0, _NSUB, idx_body, 0, unroll=False)

    def fire(bs):
        return [
            pltpu.async_copy(emb_hbm.at[idx_bufs[bs][c]], row_bufs[bs][c],
                             sems[bs])
            for c in range(8)
        ]

    def accumulate(lv, bs):
        col2 = lv * 2
        fxb, fyb, fzb = frac_bufs[bs]
        rows = row_bufs[bs]

        def acc_body(i, _):
            o16 = i * 16
            fx = fxb[pl.ds(o16, 16)]
            fy = fyb[pl.ds(o16, 16)]
            fz = fzb[pl.ds(o16, 16)]
            gx = 1.0 - fx
            gy = 1.0 - fy
            gz = 1.0 - fz
            wxy = [gx * gy, fx * gy, gx * fy, fx * fy]
            g0 = iota2 + i * 32
            acc0 = jnp.zeros((16,), jnp.float32)
            acc1 = jnp.zeros((16,), jnp.float32)
            for c in range(8):
                w = wxy[c & 3] * (fz if (c & 4) else gz)
                r0 = plsc.load_gather(rows[c], [g0])
                r1 = plsc.load_gather(rows[c], [g0 + 1])
                acc0 = acc0 + w * r0
                acc1 = acc1 + w * r1
            o0 = iota32 + (i * 512 + col2)
            plsc.store_scatter(outb, [o0], acc0)
            plsc.store_scatter(outb, [o0 + 1], acc1)
            return 0

        lax.fori_loop(0, _NSUB, acc_body, 0, unroll=False)

    def chunk_body(t, _):
        pt_base = wid * _W + t * _P
        pltpu.sync_copy(x_hbm.at[pl.ds(pt_base, _P)], xb)
        pltpu.sync_copy(y_hbm.at[pl.ds(pt_base, _P)], yb)
        pltpu.sync_copy(z_hbm.at[pl.ds(pt_base, _P)], zb)

        compute_idx(0, 0)
        pending = fire(0)
        for lv in range(_L):
            bs = lv % 2
            if lv + 1 < _L:
                compute_idx(lv + 1, 1 - bs)
                nxt = fire(1 - bs)
            for cp in pending:
                cp.wait()
            accumulate(lv, bs)
            if lv + 1 < _L:
                pending = nxt

        pltpu.sync_copy(outb, out_hbm.at[pl.ds(pt_base * _OUT_D, _P * _OUT_D)])
        return 0

    lax.fori_loop(0, _NCH, chunk_body, 0, unroll=False)


@jax.jit
def _grid_encode_sc(inputs, embeddings):
    conv = pl.kernel(
        _conv_body,
        out_type=jax.ShapeDtypeStruct((_N_ROWS * _C,), jnp.float32),
        mesh=plsc.VectorSubcoreMesh(core_axis_name="c", subcore_axis_name="s"),
        scratch_types=[
            pltpu.VMEM((_CCH,), jnp.float32),
            pltpu.VMEM((_CCH,), jnp.float32),
            pltpu.VMEM((_CCH * 2,), jnp.float32),
            pltpu.VMEM(((_CTAIL // 16 + 1) * 32,), jnp.float32),
        ],
        compiler_params=pltpu.CompilerParams(needs_layout_passes=False),
    )
    emb_flat = conv(embeddings[:, 0], embeddings[:, 1])

    scratch = [pltpu.VMEM((_P,), jnp.float32) for _ in range(3)]
    scratch += [pltpu.VMEM((_P,), jnp.float32) for _ in range(6)]
    scratch += [pltpu.VMEM((_P * _C,), jnp.int32) for _ in range(16)]
    scratch += [pltpu.VMEM((_P * _C,), jnp.float32) for _ in range(16)]
    scratch += [
        pltpu.VMEM((_P * _OUT_D,), jnp.float32),
        pltpu.SemaphoreType.DMA,
        pltpu.SemaphoreType.DMA,
    ]
    kern = pl.kernel(
        _main_body,
        out_type=jax.ShapeDtypeStruct((_B * _OUT_D,), jnp.float32),
        mesh=plsc.VectorSubcoreMesh(core_axis_name="c", subcore_axis_name="s"),
        scratch_types=scratch,
        compiler_params=pltpu.CompilerParams(needs_layout_passes=False),
    )
    flat = kern(inputs[:, 0], inputs[:, 1], inputs[:, 2], emb_flat)
    return flat.reshape(_B, _OUT_D)


def kernel(inputs, embeddings, offsets):
    del offsets
    return _grid_encode_sc(inputs, embeddings)


# speedup vs baseline: 1.6613x; 1.6572x over previous
import jax
import jax.numpy as jnp
import numpy as np
from jax import lax
from jax.experimental import pallas as pl
from jax.experimental.pallas import tpu as pltpu
from jax.experimental.pallas import tpu_sc as plsc


_B = 524288
_D = 3
_C = 2
_L = 16
_OUT_D = _L * _C
_NW = 32
_W = _B // _NW
_P = 512
_NCH = _W // _P
_NSUB = _P // 16

_N_ROWS = 7131240
_MASK = (1 << 19) - 1
_PRIME1 = np.int32(np.uint32(2654435761).view(np.int32))
_PRIME2 = np.int32(805459861)

_DENSE = [
    (15.0, 17, 17 * 17, 0),
    (31.0, 33, 33 * 33, 4920),
    (63.0, 65, 65 * 65, 40864),
]
_HASH_BASE0 = 315496
_HASH_SCALE0 = 127.0
_HASH_STRIDE = 1 << 19

_CCH = 2048
_NFULL = _N_ROWS // _CCH
_CTAIL = _N_ROWS - _NFULL * _CCH
_CREM = _NFULL % _NW


def _conv_body(ch0_hbm, ch1_hbm, out2_hbm, c0, c1, ibuf, tbuf):
    wid = lax.axis_index("s") * 2 + lax.axis_index("c")
    iota = lax.iota(jnp.int32, 16)
    c_zero = jnp.zeros((16,), jnp.int32)
    c_one = c_zero + 1

    def interleave(n16, src0, src1, dst):
        def body(i, _):
            o16 = i * 16
            v0 = src0[pl.ds(o16, 16)]
            v1 = src1[pl.ds(o16, 16)]
            pos = iota + o16
            plsc.store_scatter(dst, [pos, c_zero], v0)
            plsc.store_scatter(dst, [pos, c_one], v1)
            return 0
        lax.fori_loop(0, n16, body, 0, unroll=False)

    ntrips = 108 + (wid < _CREM).astype(jnp.int32)

    def chunk(k, _):
        b = (k * _NW + wid) * _CCH
        pltpu.sync_copy(ch0_hbm.at[pl.ds(b, _CCH)], c0)
        pltpu.sync_copy(ch1_hbm.at[pl.ds(b, _CCH)], c1)
        interleave(_CCH // 16, c0, c1, ibuf)
        pltpu.sync_copy(ibuf, out2_hbm.at[pl.ds(b, _CCH), :])
        return 0

    lax.fori_loop(0, ntrips, chunk, 0, unroll=False)

    @pl.when(wid == _NW - 1)
    def _():
        b = _NFULL * _CCH
        pltpu.sync_copy(ch0_hbm.at[pl.ds(b, _CTAIL)], c0.at[pl.ds(0, _CTAIL)])
        pltpu.sync_copy(ch1_hbm.at[pl.ds(b, _CTAIL)], c1.at[pl.ds(0, _CTAIL)])
        interleave(_CTAIL // 16, c0, c1, tbuf)
        o16 = (_CTAIL // 16) * 16
        v0 = c0[pl.ds(o16, 16)]
        v1 = c1[pl.ds(o16, 16)]
        pos = iota + o16
        msk = iota < (_CTAIL - o16)
        plsc.store_scatter(tbuf, [pos, c_zero], v0, mask=msk)
        plsc.store_scatter(tbuf, [pos, c_one], v1, mask=msk)
        pltpu.sync_copy(tbuf.at[pl.ds(0, _CTAIL), :],
                        out2_hbm.at[pl.ds(b, _CTAIL), :])


def _main_body(x_hbm, y_hbm, z_hbm, emb_hbm, out_hbm, *scratch):
    xb, yb, zb = scratch[0:3]
    frac_bufs = (scratch[3:6], scratch[6:9])
    idx_bufs = (scratch[9:17], scratch[17:25])
    row_bufs = (scratch[25:33], scratch[33:41])
    outb = scratch[41]
    sems = (scratch[42], scratch[43])

    wid = lax.axis_index("s") * 2 + lax.axis_index("c")

    iota = lax.iota(jnp.int32, 16)
    iota2 = iota * 2
    iota32 = iota * 32

    levels = []
    for l, (scale, r, r2, base) in enumerate(_DENSE):
        levels.append((scale, base, (r, r2, False)))
    for l in range(3, 12):
        levels.append((2.0 ** l * 16 - 1, _HASH_BASE0 + (l - 3) * _HASH_STRIDE,
                       None))
    levels.append((65535.0, 5034088, (65537, 131073, True)))
    levels.append((131071.0, 5558376, (131073, 262145, True)))
    levels.append((262143.0, 6082664, None))
    levels.append((524287.0, 6606952, None))

    def compute_idx(lv, bs):
        scale, base, lin = levels[lv]
        s_f = jnp.float32(scale)
        base = jnp.int32(base)
        fxb, fyb, fzb = frac_bufs[bs]
        idxs = idx_bufs[bs]

        def idx_body(i, _):
            o16 = i * 16
            x = xb[pl.ds(o16, 16)]
            y = yb[pl.ds(o16, 16)]
            z = zb[pl.ds(o16, 16)]

            px = x * s_f + 0.5
            py = y * s_f + 0.5
            pz = z * s_f + 0.5
            ix0 = px.astype(jnp.int32)
            iy0 = py.astype(jnp.int32)
            iz0 = pz.astype(jnp.int32)
            fxb[pl.ds(o16, 16)] = px - ix0.astype(jnp.float32)
            fyb[pl.ds(o16, 16)] = py - iy0.astype(jnp.float32)
            fzb[pl.ds(o16, 16)] = pz - iz0.astype(jnp.float32)
            ix1 = ix0 + 1
            if lin is None:
                hy0 = iy0 * _PRIME1
                hy1 = hy0 + _PRIME1
                hz0 = iz0 * _PRIME2
                hz1 = hz0 + _PRIME2
                terms = []
                for c in range(8):
                    xx = ix1 if (c & 1) else ix0
                    hy = hy1 if (c & 2) else hy0
                    hz = hz1 if (c & 4) else hz0
                    terms.append(((xx ^ hy ^ hz) & _MASK) + base)
            else:
                m1, m2, masked = lin
                my0 = iy0 * m1
                my1 = my0 + m1
                mz0 = iz0 * m2
                mz1 = mz0 + m2
                terms = []
                for c in range(8):
                    xx = ix1 if (c & 1) else ix0
                    my = my1 if (c & 2) else my0
                    mz = mz1 if (c & 4) else mz0
                    t = xx + my + mz
                    if masked:
                        t = t & _MASK
                    terms.append(t + base)
            for c in range(8):
                idxs[c][pl.ds(o16, 16)] = terms[c]
            return 0

        lax.fori_loop(0, _NSUB, idx_body, 0, unroll=False)

    def fire(bs):
        return [
            pltpu.async_copy(emb_hbm.at[idx_bufs[bs][c]], row_bufs[bs][c],
                             sems[bs])
            for c in range(8)
        ]

    def accumulate(lv, bs):
        col2 = lv * 2
        fxb, fyb, fzb = frac_bufs[bs]
        rows = row_bufs[bs]

        def acc_body(i, _):
            o16 = i * 16
            fx = fxb[pl.ds(o16, 16)]
            fy = fyb[pl.ds(o16, 16)]
            fz = fzb[pl.ds(o16, 16)]
            gx = 1.0 - fx
            gy = 1.0 - fy
            gz = 1.0 - fz
            wxy = [gx * gy, fx * gy, gx * fy, fx * fy]
            ridx = iota + o16
            c_zero = jnp.zeros((16,), jnp.int32)
            c_one = c_zero + 1
            acc0 = jnp.zeros((16,), jnp.float32)
            acc1 = jnp.zeros((16,), jnp.float32)
            for c in range(8):
                w = wxy[c & 3] * (fz if (c & 4) else gz)
                r0 = plsc.load_gather(rows[c], [ridx, c_zero])
                r1 = plsc.load_gather(rows[c], [ridx, c_one])
                acc0 = acc0 + w * r0
                acc1 = acc1 + w * r1
            o0 = iota32 + (i * 512 + col2)
            plsc.store_scatter(outb, [o0], acc0)
            plsc.store_scatter(outb, [o0 + 1], acc1)
            return 0

        lax.fori_loop(0, _NSUB, acc_body, 0, unroll=False)

    def chunk_body(t, _):
        pt_base = wid * _W + t * _P
        pltpu.sync_copy(x_hbm.at[pl.ds(pt_base, _P)], xb)
        pltpu.sync_copy(y_hbm.at[pl.ds(pt_base, _P)], yb)
        pltpu.sync_copy(z_hbm.at[pl.ds(pt_base, _P)], zb)

        compute_idx(0, 0)
        pending = fire(0)
        for lv in range(_L):
            bs = lv % 2
            if lv + 1 < _L:
                compute_idx(lv + 1, 1 - bs)
                nxt = fire(1 - bs)
            for cp in pending:
                cp.wait()
            accumulate(lv, bs)
            if lv + 1 < _L:
                pending = nxt

        pltpu.sync_copy(outb, out_hbm.at[pl.ds(pt_base * _OUT_D, _P * _OUT_D)])
        return 0

    lax.fori_loop(0, _NCH, chunk_body, 0, unroll=False)


@jax.jit
def _grid_encode_sc(inputs, embeddings):
    conv = pl.kernel(
        _conv_body,
        out_type=jax.ShapeDtypeStruct((_N_ROWS, _C), jnp.float32),
        mesh=plsc.VectorSubcoreMesh(core_axis_name="c", subcore_axis_name="s"),
        scratch_types=[
            pltpu.VMEM((_CCH,), jnp.float32),
            pltpu.VMEM((_CCH,), jnp.float32),
            pltpu.VMEM((_CCH, _C), jnp.float32),
            pltpu.VMEM(((_CTAIL // 16 + 1) * 16, _C), jnp.float32),
        ],
        compiler_params=pltpu.CompilerParams(needs_layout_passes=False,
                                             use_tc_tiling_on_sc=False),
    )
    emb_flat = conv(embeddings[:, 0], embeddings[:, 1])

    scratch = [pltpu.VMEM((_P,), jnp.float32) for _ in range(3)]
    scratch += [pltpu.VMEM((_P,), jnp.float32) for _ in range(6)]
    scratch += [pltpu.VMEM((_P,), jnp.int32) for _ in range(16)]
    scratch += [pltpu.VMEM((_P, _C), jnp.float32) for _ in range(16)]
    scratch += [
        pltpu.VMEM((_P * _OUT_D,), jnp.float32),
        pltpu.SemaphoreType.DMA,
        pltpu.SemaphoreType.DMA,
    ]
    kern = pl.kernel(
        _main_body,
        out_type=jax.ShapeDtypeStruct((_B * _OUT_D,), jnp.float32),
        mesh=plsc.VectorSubcoreMesh(core_axis_name="c", subcore_axis_name="s"),
        scratch_types=scratch,
        compiler_params=pltpu.CompilerParams(needs_layout_passes=False,
                                             use_tc_tiling_on_sc=False),
    )
    flat = kern(inputs[:, 0], inputs[:, 1], inputs[:, 2], emb_flat)
    return flat.reshape(_B, _OUT_D)


def kernel(inputs, embeddings, offsets):
    del offsets
    return _grid_encode_sc(inputs, embeddings)
